# baseline (speedup 1.0000x reference)
import functools

import jax
import jax.numpy as jnp
from jax.experimental import pallas as pl
from jax.experimental.pallas import tpu as pltpu

NCODES = 8192
DIM = 256
ROWS_PER_BLOCK = 512
SUBROWS = 128
SEGMENTS = ((0, 2736), (2736, 5472), (5472, NCODES))
SEGPAD = 2816
CBPAD = SEGPAD - (NCODES - SEGMENTS[-1][0])


def _seg_scan(z2r, e2p, zer, lo):
    m = None
    ix = None
    ids0 = jax.lax.broadcasted_iota(jnp.int32, (SUBROWS, 128), 1) + lo
    for k in range(SEGPAD // 128):
        e2k = jax.lax.slice(e2p, (0, k * 128), (1, (k + 1) * 128))
        zek = jax.lax.slice(zer, (0, k * 128), (SUBROWS, (k + 1) * 128))
        t = (z2r + e2k) + zek
        ids = ids0 + k * 128
        if m is None:
            m, ix = t, ids
        else:
            take = t < m
            m = jnp.where(take, t, m)
            ix = jnp.where(take, ids, ix)
    cmin = jnp.min(m, axis=1, keepdims=True)
    cidx = jnp.min(jnp.where(m == cmin, ix, NCODES), axis=1, keepdims=True)
    return cmin, cidx


def _dist_argmin_body(z_ref, cb_hbm, idx_ref, md_ref, cb_vmem, e2_ref, sem):
    @pl.when(pl.program_id(0) == 0)
    def _setup():
        copy = pltpu.make_async_copy(cb_hbm, cb_vmem.at[pl.ds(0, NCODES), :],
                                     sem)
        copy.start()
        copy.wait()
        cb_vmem[pl.ds(NCODES, CBPAD), :] = jnp.zeros((CBPAD, DIM), jnp.float32)
        e2_ref[...] = jnp.full((len(SEGMENTS), SEGPAD), 1e30, jnp.float32)
        for s, (lo, hi) in enumerate(SEGMENTS):
            cb = cb_vmem[pl.ds(lo, hi - lo), :]
            e2_ref[pl.ds(s, 1), :hi - lo] = jnp.sum(cb * cb, axis=1)[None, :]
        cb_vmem[pl.ds(0, NCODES), :] = cb_vmem[pl.ds(0, NCODES), :] * -2.0

    zb = z_ref[...]
    z2 = jnp.sum(zb * zb, axis=1, keepdims=True)
    per_seg = []
    for s, (lo, hi) in enumerate(SEGMENTS):
        cb2 = cb_vmem[pl.ds(lo, SEGPAD), :]
        ze2 = jax.lax.dot_general(
            zb, cb2, (((1,), (1,)), ((), ())),
            preferred_element_type=jnp.float32)
        e2p = e2_ref[pl.ds(s, 1), :]
        cmins, cidxs = [], []
        for r in range(ROWS_PER_BLOCK // SUBROWS):
            z2r = jax.lax.slice(z2, (r * SUBROWS, 0), ((r + 1) * SUBROWS, 1))
            zer = jax.lax.slice(ze2, (r * SUBROWS, 0),
                                ((r + 1) * SUBROWS, SEGPAD))
            cmin, cidx = _seg_scan(z2r, e2p, zer, lo)
            cmins.append(cmin)
            cidxs.append(cidx)
        per_seg.append((jnp.concatenate(cmins, axis=0),
                        jnp.concatenate(cidxs, axis=0)))

    minv = None
    mind = None
    mini = None
    for cmin, cidx in per_seg:
        if minv is None:
            minv, mind, mini = cmin, cmin, cidx
        else:
            spilled = minv.astype(jnp.bfloat16).astype(jnp.float32)
            upd = cmin < spilled
            minv = jnp.where(upd, cmin, spilled)
            mind = jnp.where(upd, cmin, mind)
            mini = jnp.where(upd, cidx, mini)
    idx_ref[0] = mini
    md_ref[0] = mind


def kernel(z, codebook):
    B, Dd, H, W = z.shape
    N = B * H * W
    z_flat = jnp.transpose(z, (0, 2, 3, 1)).reshape(N, Dd)
    nb = N // ROWS_PER_BLOCK
    idx3, md3 = pl.pallas_call(
        _dist_argmin_body,
        grid=(nb,),
        in_specs=[
            pl.BlockSpec((ROWS_PER_BLOCK, Dd), lambda i: (i, 0)),
            pl.BlockSpec(memory_space=pltpu.MemorySpace.HBM),
        ],
        out_specs=[
            pl.BlockSpec((1, ROWS_PER_BLOCK, 1), lambda i: (i, 0, 0)),
            pl.BlockSpec((1, ROWS_PER_BLOCK, 1), lambda i: (i, 0, 0)),
        ],
        out_shape=[
            jax.ShapeDtypeStruct((nb, ROWS_PER_BLOCK, 1), jnp.int32),
            jax.ShapeDtypeStruct((nb, ROWS_PER_BLOCK, 1), jnp.float32),
        ],
        scratch_shapes=[
            pltpu.VMEM((NCODES + CBPAD, DIM), jnp.float32),
            pltpu.VMEM((len(SEGMENTS), SEGPAD), jnp.float32),
            pltpu.SemaphoreType.DMA,
        ],
    )(z_flat, codebook)
    idx = idx3.reshape(N)
    vq_loss = 1.25 * (jnp.sum(md3) / (N * Dd))
    z_q_rows = jnp.take(codebook, idx, axis=0)
    z_q = jnp.transpose(z_q_rows.reshape(B, H, W, Dd), (0, 3, 1, 2))
    return (z_q, vq_loss, idx.reshape(B, H, W))


# speedup vs baseline: 1.3158x; 1.3158x over previous
import functools

import jax
import jax.numpy as jnp
from jax import lax
from jax.experimental import pallas as pl
from jax.experimental.pallas import tpu as pltpu
from jax.experimental.pallas import tpu_sc as plsc

NCODES = 8192
DIM = 256
ROWS_PER_BLOCK = 512
SUBROWS = 128
SEGMENTS = ((0, 2736), (2736, 5472), (5472, NCODES))
SEGPAD = 2816
CBPAD = SEGPAD - (NCODES - SEGMENTS[-1][0])


def _seg_scan(z2r, e2p, zer, lo):
    m = None
    ix = None
    ids0 = jax.lax.broadcasted_iota(jnp.int32, (SUBROWS, 128), 1) + lo
    for k in range(SEGPAD // 128):
        e2k = jax.lax.slice(e2p, (0, k * 128), (1, (k + 1) * 128))
        zek = jax.lax.slice(zer, (0, k * 128), (SUBROWS, (k + 1) * 128))
        t = (z2r + e2k) + zek
        ids = ids0 + k * 128
        if m is None:
            m, ix = t, ids
        else:
            take = t < m
            m = jnp.where(take, t, m)
            ix = jnp.where(take, ids, ix)
    cmin = jnp.min(m, axis=1, keepdims=True)
    cidx = jnp.min(jnp.where(m == cmin, ix, NCODES), axis=1, keepdims=True)
    return cmin, cidx


def _dist_argmin_body(z_ref, cb_hbm, idx_ref, md_ref, cb_vmem, e2_ref, sem):
    @pl.when(pl.program_id(0) == 0)
    def _setup():
        copy = pltpu.make_async_copy(cb_hbm, cb_vmem.at[pl.ds(0, NCODES), :],
                                     sem)
        copy.start()
        copy.wait()
        cb_vmem[pl.ds(NCODES, CBPAD), :] = jnp.zeros((CBPAD, DIM), jnp.float32)
        e2_ref[...] = jnp.full((len(SEGMENTS), SEGPAD), 1e30, jnp.float32)
        for s, (lo, hi) in enumerate(SEGMENTS):
            cb = cb_vmem[pl.ds(lo, hi - lo), :]
            e2_ref[pl.ds(s, 1), :hi - lo] = jnp.sum(cb * cb, axis=1)[None, :]
        cb_vmem[pl.ds(0, NCODES), :] = cb_vmem[pl.ds(0, NCODES), :] * -2.0

    zb = z_ref[...]
    z2 = jnp.sum(zb * zb, axis=1, keepdims=True)
    per_seg = []
    for s, (lo, hi) in enumerate(SEGMENTS):
        cb2 = cb_vmem[pl.ds(lo, SEGPAD), :]
        ze2 = jax.lax.dot_general(
            zb, cb2, (((1,), (1,)), ((), ())),
            preferred_element_type=jnp.float32)
        e2p = e2_ref[pl.ds(s, 1), :]
        cmins, cidxs = [], []
        for r in range(ROWS_PER_BLOCK // SUBROWS):
            z2r = jax.lax.slice(z2, (r * SUBROWS, 0), ((r + 1) * SUBROWS, 1))
            zer = jax.lax.slice(ze2, (r * SUBROWS, 0),
                                ((r + 1) * SUBROWS, SEGPAD))
            cmin, cidx = _seg_scan(z2r, e2p, zer, lo)
            cmins.append(cmin)
            cidxs.append(cidx)
        per_seg.append((jnp.concatenate(cmins, axis=0),
                        jnp.concatenate(cidxs, axis=0)))

    minv = None
    mind = None
    mini = None
    for cmin, cidx in per_seg:
        if minv is None:
            minv, mind, mini = cmin, cmin, cidx
        else:
            spilled = minv.astype(jnp.bfloat16).astype(jnp.float32)
            upd = cmin < spilled
            minv = jnp.where(upd, cmin, spilled)
            mind = jnp.where(upd, cmin, mind)
            mini = jnp.where(upd, cidx, mini)
    idx_ref[0] = mini
    md_ref[0] = mind


GATHER_CHUNK = 256


def _sc_gather_body(table_hbm, idx_hbm, out_hbm, idx_v, rows_v, sem):
    n_chunks = 32768 // (32 * GATHER_CHUNK)
    wid = lax.axis_index("s") * 2 + lax.axis_index("c")
    for c in range(n_chunks):
        base = (wid * n_chunks + c) * GATHER_CHUNK
        pltpu.sync_copy(idx_hbm.at[pl.ds(base, GATHER_CHUNK)], idx_v)
        pltpu.async_copy(table_hbm.at[idx_v], rows_v, sem).wait()
        pltpu.sync_copy(rows_v, out_hbm.at[pl.ds(base, GATHER_CHUNK)])


def _sc_gather(codebook, idx):
    mesh = plsc.VectorSubcoreMesh(core_axis_name="c", subcore_axis_name="s")
    return pl.kernel(
        _sc_gather_body,
        out_type=jax.ShapeDtypeStruct((idx.shape[0], DIM), jnp.float32),
        mesh=mesh,
        scratch_types=[
            pltpu.VMEM((GATHER_CHUNK,), jnp.int32),
            pltpu.VMEM((GATHER_CHUNK, DIM), jnp.float32),
            pltpu.SemaphoreType.DMA,
        ],
    )(codebook, idx)


def kernel(z, codebook):
    B, Dd, H, W = z.shape
    N = B * H * W
    z_flat = jnp.transpose(z, (0, 2, 3, 1)).reshape(N, Dd)
    nb = N // ROWS_PER_BLOCK
    idx3, md3 = pl.pallas_call(
        _dist_argmin_body,
        grid=(nb,),
        in_specs=[
            pl.BlockSpec((ROWS_PER_BLOCK, Dd), lambda i: (i, 0)),
            pl.BlockSpec(memory_space=pltpu.MemorySpace.HBM),
        ],
        out_specs=[
            pl.BlockSpec((1, ROWS_PER_BLOCK, 1), lambda i: (i, 0, 0)),
            pl.BlockSpec((1, ROWS_PER_BLOCK, 1), lambda i: (i, 0, 0)),
        ],
        out_shape=[
            jax.ShapeDtypeStruct((nb, ROWS_PER_BLOCK, 1), jnp.int32),
            jax.ShapeDtypeStruct((nb, ROWS_PER_BLOCK, 1), jnp.float32),
        ],
        scratch_shapes=[
            pltpu.VMEM((NCODES + CBPAD, DIM), jnp.float32),
            pltpu.VMEM((len(SEGMENTS), SEGPAD), jnp.float32),
            pltpu.SemaphoreType.DMA,
        ],
    )(z_flat, codebook)
    idx = idx3.reshape(N)
    vq_loss = 1.25 * (jnp.sum(md3) / (N * Dd))
    z_q_rows = _sc_gather(codebook, idx)
    z_q = jnp.transpose(z_q_rows.reshape(B, H, W, Dd), (0, 3, 1, 2))
    return (z_q, vq_loss, idx.reshape(B, H, W))
